# baseline (speedup 1.0000x reference)
import jax
import jax.numpy as jnp
from jax import lax
from jax.experimental import pallas as pl
from jax.experimental.pallas import tpu as pltpu
from jax.experimental.pallas import tpu_sc as plsc

N_NODES = 10000
N_RELS = 500
N_EDGES = 320000
D = 128

NC = 2
NS = 16
L = 16

HALF = N_NODES // NC
ROWS_A = 10240
ROWS_B = 5120
RELBASE = 5632

EPT = N_EDGES // NS
SB = 800
NSB = EPT // SB
CH = 80
NCH = SB // CH


def _sc_body(ent_hbm, rel_hbm, src_hbm, dst_hbm, typ_hbm, dir_hbm, nrm_hbm,
             acca_hbm, accb_hbm,
             src_v, dst_v, typ_v, dir_v, nrm_v, coff, cscl, oidx,
             ent_rows, rel_rows, acc_sh,
             sem_e0, sem_e1, sem_r0, sem_r1, sem_s0, sem_s1, sem_m):
    core = lax.axis_index("c")
    tid = lax.axis_index("s")
    lo = core * HALF
    zeros16 = jnp.zeros((L,), jnp.float32)
    sem_e = (sem_e0, sem_e1)
    sem_r = (sem_r0, sem_r1)
    sem_s = (sem_s0, sem_s1)

    def _zrow(r, _):
        for q in range(D // L):
            ent_rows[0, r, pl.ds(q * L, L)] = zeros16
        return 0

    def run_pass(acc_rows, out_at, mark_fn, rel_in_sh):
        rows_per_tile = acc_rows // NS
        trash = acc_rows - 1

        lax.fori_loop(0, CH, _zrow, 0)
        for k in range(rows_per_tile // CH):
            pltpu.sync_copy(
                ent_rows.at[0],
                acc_sh.at[pl.ds(tid * rows_per_tile + k * CH, CH)])
        if rel_in_sh:
            @pl.when(tid < NS - 1)
            def _stage():
                pltpu.sync_copy(
                    rel_hbm.at[pl.ds(tid * 32, 32)],
                    acc_sh.at[pl.ds(RELBASE + tid * 32, 32)])

            @pl.when(tid == NS - 1)
            def _stage_tail():
                pltpu.sync_copy(
                    rel_hbm.at[pl.ds(480, 20)],
                    acc_sh.at[pl.ds(RELBASE + 480, 20)])
        plsc.subcore_barrier()

        def _gather(c, base):
            b = c % 2
            cb = c * CH
            ge = pltpu.async_copy(ent_hbm.at[src_v.at[pl.ds(cb, CH)]],
                                  ent_rows.at[b], sem_e[b])
            if rel_in_sh:
                gr = pltpu.async_copy(acc_sh.at[dir_v.at[pl.ds(cb, CH)]],
                                      rel_rows.at[b], sem_r[b])
            else:
                gr = pltpu.async_copy(rel_hbm.at[typ_v.at[pl.ds(cb, CH)]],
                                      rel_rows.at[b], sem_r[b])
            return ge, gr

        def _superblock(sb, _):
            base = tid * EPT + sb * SB
            hs = [pltpu.async_copy(h.at[pl.ds(base, SB)], v, sem_m)
                  for h, v in ((src_hbm, src_v), (dst_hbm, dst_v),
                               (typ_hbm, typ_v), (dir_hbm, dir_v),
                               (nrm_hbm, nrm_v))]
            for h in hs:
                h.wait()

            def _meta(g, _):
                p = g * L
                d16 = dir_v[pl.ds(p, L)]
                dst16 = dst_v[pl.ds(p, L)]
                n16 = nrm_v[pl.ds(p, L)]
                own, off16 = mark_fn(d16, dst16)
                own = own & (n16 != 0.0)
                coff[pl.ds(p, L)] = jnp.where(own, off16, trash)
                cscl[pl.ds(p, L)] = jnp.where(own, n16, 0.0)
                if rel_in_sh:
                    dir_v[pl.ds(p, L)] = typ_v[pl.ds(p, L)] + RELBASE
                return 0

            lax.fori_loop(0, SB // L, _meta, 0)

            gath = _gather(0, base)
            scat = [None, None]
            for c in range(NCH):
                b = c % 2
                cb = c * CH
                if c + 1 < NCH:
                    if scat[1 - b] is not None:
                        scat[1 - b].wait()
                        scat[1 - b] = None
                    nxt = _gather(c + 1, base)
                gath[0].wait()
                gath[1].wait()

                def _egroup(g, _):
                    s16 = cscl[pl.ds(cb + g * L, L)]
                    oidx[b, pl.ds(g * L, L)] = coff[pl.ds(cb + g * L, L)]
                    for j in range(L):
                        e = g * L + j
                        sj = jnp.broadcast_to(s16[j], (L,))

                        @pl.when(s16[j] != 0.0)
                        def _scale():
                            for q in range(D // L):
                                sl = pl.ds(q * L, L)
                                ent_rows[b, e, sl] = (
                                    ent_rows[b, e, sl]
                                    - rel_rows[b, e, sl]) * sj
                    return 0

                lax.fori_loop(0, CH // L, _egroup, 0)
                scat[b] = pltpu.async_copy(ent_rows.at[b],
                                           acc_sh.at[oidx.at[b]], sem_s[b],
                                           add=True)
                if c + 1 < NCH:
                    gath = nxt
            for s in scat:
                if s is not None:
                    s.wait()
            return 0

        lax.fori_loop(0, NSB, _superblock, 0)
        plsc.subcore_barrier()

        r0 = tid * rows_per_tile
        pltpu.sync_copy(acc_sh.at[pl.ds(r0, rows_per_tile)],
                        out_at.at[pl.ds(r0, rows_per_tile)])
        plsc.subcore_barrier()

    def _mark_a(d16, dst16):
        return d16 == core, dst16

    def _mark_b(d16, dst16):
        off16 = dst16 - lo
        return (d16 == 2) & (off16 >= 0) & (off16 < HALF), off16

    run_pass(ROWS_A, acca_hbm.at[core], _mark_a, False)
    run_pass(ROWS_B, accb_hbm.at[core], _mark_b, True)


def _sc_aggregate(ent_emb, rel_emb, src, dst, etype, edir, enorm):
    mesh = plsc.VectorSubcoreMesh(core_axis_name="c", subcore_axis_name="s",
                                  num_cores=NC, num_subcores=NS)
    f = pl.kernel(
        _sc_body,
        out_type=(jax.ShapeDtypeStruct((NC, ROWS_A, D), jnp.float32),
                  jax.ShapeDtypeStruct((NC, ROWS_B, D), jnp.float32)),
        mesh=mesh,
        scratch_types=[
            pltpu.VMEM((SB,), jnp.int32),
            pltpu.VMEM((SB,), jnp.int32),
            pltpu.VMEM((SB,), jnp.int32),
            pltpu.VMEM((SB,), jnp.int32),
            pltpu.VMEM((SB,), jnp.float32),
            pltpu.VMEM((SB,), jnp.int32),
            pltpu.VMEM((SB,), jnp.float32),
            pltpu.VMEM((2, CH), jnp.int32),
            pltpu.VMEM((2, CH, D), jnp.float32),
            pltpu.VMEM((2, CH, D), jnp.float32),
            pltpu.VMEM_SHARED((ROWS_A, D), jnp.float32),
            pltpu.SemaphoreType.DMA,
            pltpu.SemaphoreType.DMA,
            pltpu.SemaphoreType.DMA,
            pltpu.SemaphoreType.DMA,
            pltpu.SemaphoreType.DMA,
            pltpu.SemaphoreType.DMA,
            pltpu.SemaphoreType.DMA,
        ],
    )
    return f(ent_emb, rel_emb, src, dst, etype, edir, enorm)


def _tc_body(acca_ref, accb_ref, w_ref, rel_ref, wrel_ref, gam_ref, bet_ref,
             out_ref, relout_ref):
    acc0 = acca_ref[0, :N_NODES, :]
    acc1 = acca_ref[1, :N_NODES, :]
    acc2 = jnp.concatenate([accb_ref[0, :HALF, :], accb_ref[1, :HALF, :]],
                           axis=0)
    y = jnp.dot(acc0, w_ref[0], preferred_element_type=jnp.float32)
    y = y + jnp.dot(acc1, w_ref[1], preferred_element_type=jnp.float32)
    y = y + jnp.dot(acc2, w_ref[2], preferred_element_type=jnp.float32)
    mean = jnp.mean(y, axis=0, keepdims=True)
    var = jnp.mean((y - mean) * (y - mean), axis=0, keepdims=True)
    yn = (y - mean) * lax.rsqrt(var + 1e-5) * gam_ref[...] + bet_ref[...]
    out_ref[...] = jnp.maximum(yn, 0.0)
    relout_ref[...] = jnp.maximum(
        jnp.dot(rel_ref[...], wrel_ref[...],
                preferred_element_type=jnp.float32), 0.0)


def _tc_finish(acca, accb, w_stack, rel_emb, w_rel_t, gamma, beta):
    return pl.pallas_call(
        _tc_body,
        out_shape=(jax.ShapeDtypeStruct((N_NODES, D), jnp.float32),
                   jax.ShapeDtypeStruct((N_RELS, D), jnp.float32)),
    )(acca, accb, w_stack, rel_emb, w_rel_t, gamma, beta)


def kernel(ent_emb, rel_emb, edge_norm, W_in, W_out, W_loop, W_rel, gamma,
           beta, edge_index, edge_type, edge_dir):
    src = edge_index[0].astype(jnp.int32)
    dst = edge_index[1].astype(jnp.int32)
    etype = edge_type.astype(jnp.int32)
    edir = edge_dir.astype(jnp.int32)

    acca, accb = _sc_aggregate(ent_emb, rel_emb, src, dst, etype, edir,
                               edge_norm)

    w_stack = jnp.stack([W_out.T, W_in.T, W_loop.T])
    out, rel_out = _tc_finish(acca, accb, w_stack, rel_emb, W_rel.T,
                              gamma.reshape(1, D), beta.reshape(1, D))
    return (out, rel_out)


# speedup vs baseline: 1.1407x; 1.1407x over previous
import jax
import jax.numpy as jnp
from jax import lax
from jax.experimental import pallas as pl
from jax.experimental.pallas import tpu as pltpu
from jax.experimental.pallas import tpu_sc as plsc

N_NODES = 10000
N_RELS = 500
N_EDGES = 320000
D = 128

NC = 2
NS = 16
L = 16

HALF = N_NODES // NC
ROWS_A = 10240
ROWS_B = 5120
RELBASE = 5632

EPT = N_EDGES // NS
SB = 800
NSB = EPT // SB
CH = 80
NCH = SB // CH


def _sc_body(ent_hbm, rel_hbm, src_hbm, dst_hbm, typ_hbm, dir_hbm, nrm_hbm,
             acca_hbm, accb_hbm,
             src_v, dst_v, typ_v, dir_v, nrm_v, coff, cscl, oidx,
             ent_rows, rel_rows, acc_sh,
             sem_e0, sem_e1, sem_r0, sem_r1, sem_s0, sem_s1, sem_m):
    core = lax.axis_index("c")
    tid = lax.axis_index("s")
    lo = core * HALF
    zeros16 = jnp.zeros((L,), jnp.float32)
    sem_e = (sem_e0, sem_e1)
    sem_r = (sem_r0, sem_r1)
    sem_s = (sem_s0, sem_s1)

    def _zrow(r, _):
        for q in range(D // L):
            ent_rows[0, r, pl.ds(q * L, L)] = zeros16
        return 0

    lane16 = lax.broadcasted_iota(jnp.int32, (L,), 0)

    def run_pass(acc_rows, out_at, mark_fn, rel_in_sh, used_rows, tspan):
        rows_per_tile = acc_rows // NS
        trash = used_rows + tid * tspan + lane16 % tspan

        lax.fori_loop(0, CH, _zrow, 0)
        for k in range(rows_per_tile // CH):
            pltpu.sync_copy(
                ent_rows.at[0],
                acc_sh.at[pl.ds(tid * rows_per_tile + k * CH, CH)])
        if rel_in_sh:
            @pl.when(tid < NS - 1)
            def _stage():
                pltpu.sync_copy(
                    rel_hbm.at[pl.ds(tid * 32, 32)],
                    acc_sh.at[pl.ds(RELBASE + tid * 32, 32)])

            @pl.when(tid == NS - 1)
            def _stage_tail():
                pltpu.sync_copy(
                    rel_hbm.at[pl.ds(480, 20)],
                    acc_sh.at[pl.ds(RELBASE + 480, 20)])
        plsc.subcore_barrier()

        def _gather(c, base):
            b = c % 2
            cb = c * CH
            ge = pltpu.async_copy(ent_hbm.at[src_v.at[pl.ds(cb, CH)]],
                                  ent_rows.at[b], sem_e[b])
            if rel_in_sh:
                gr = pltpu.async_copy(acc_sh.at[dir_v.at[pl.ds(cb, CH)]],
                                      rel_rows.at[b], sem_r[b])
            else:
                gr = pltpu.async_copy(rel_hbm.at[typ_v.at[pl.ds(cb, CH)]],
                                      rel_rows.at[b], sem_r[b])
            return ge, gr

        def _superblock(sb, _):
            base = tid * EPT + sb * SB
            hs = [pltpu.async_copy(h.at[pl.ds(base, SB)], v, sem_m)
                  for h, v in ((src_hbm, src_v), (dst_hbm, dst_v),
                               (typ_hbm, typ_v), (dir_hbm, dir_v),
                               (nrm_hbm, nrm_v))]
            for h in hs:
                h.wait()

            def _meta(g, _):
                p = g * L
                d16 = dir_v[pl.ds(p, L)]
                dst16 = dst_v[pl.ds(p, L)]
                n16 = nrm_v[pl.ds(p, L)]
                own, off16 = mark_fn(d16, dst16)
                own = own & (n16 != 0.0)
                coff[pl.ds(p, L)] = jnp.where(own, off16, trash)
                cscl[pl.ds(p, L)] = jnp.where(own, n16, 0.0)
                if rel_in_sh:
                    dir_v[pl.ds(p, L)] = typ_v[pl.ds(p, L)] + RELBASE
                return 0

            lax.fori_loop(0, SB // L, _meta, 0)

            gath = _gather(0, base)
            scat = [None, None]
            for c in range(NCH):
                b = c % 2
                cb = c * CH
                if c + 1 < NCH:
                    if scat[1 - b] is not None:
                        scat[1 - b].wait()
                        scat[1 - b] = None
                    nxt = _gather(c + 1, base)
                gath[0].wait()
                gath[1].wait()

                def _egroup(g, _):
                    s16 = cscl[pl.ds(cb + g * L, L)]
                    oidx[b, pl.ds(g * L, L)] = coff[pl.ds(cb + g * L, L)]
                    for j in range(L):
                        e = g * L + j
                        sj = jnp.broadcast_to(s16[j], (L,))

                        @pl.when(s16[j] != 0.0)
                        def _scale():
                            for q in range(D // L):
                                sl = pl.ds(q * L, L)
                                ent_rows[b, e, sl] = (
                                    ent_rows[b, e, sl]
                                    - rel_rows[b, e, sl]) * sj
                    return 0

                lax.fori_loop(0, CH // L, _egroup, 0)
                scat[b] = pltpu.async_copy(ent_rows.at[b],
                                           acc_sh.at[oidx.at[b]], sem_s[b],
                                           add=True)
                if c + 1 < NCH:
                    gath = nxt
            for s in scat:
                if s is not None:
                    s.wait()
            return 0

        lax.fori_loop(0, NSB, _superblock, 0)
        plsc.subcore_barrier()

        r0 = tid * rows_per_tile
        pltpu.sync_copy(acc_sh.at[pl.ds(r0, rows_per_tile)],
                        out_at.at[pl.ds(r0, rows_per_tile)])
        plsc.subcore_barrier()

    def _mark_a(d16, dst16):
        return d16 == core, dst16

    def _mark_b(d16, dst16):
        off16 = dst16 - lo
        return (d16 == 2) & (off16 >= 0) & (off16 < HALF), off16

    run_pass(ROWS_A, acca_hbm.at[core], _mark_a, False, N_NODES, 15)
    run_pass(ROWS_B, accb_hbm.at[core], _mark_b, False, HALF, 7)


def _sc_aggregate(ent_emb, rel_emb, src, dst, etype, edir, enorm):
    mesh = plsc.VectorSubcoreMesh(core_axis_name="c", subcore_axis_name="s",
                                  num_cores=NC, num_subcores=NS)
    f = pl.kernel(
        _sc_body,
        out_type=(jax.ShapeDtypeStruct((NC, ROWS_A, D), jnp.float32),
                  jax.ShapeDtypeStruct((NC, ROWS_B, D), jnp.float32)),
        mesh=mesh,
        scratch_types=[
            pltpu.VMEM((SB,), jnp.int32),
            pltpu.VMEM((SB,), jnp.int32),
            pltpu.VMEM((SB,), jnp.int32),
            pltpu.VMEM((SB,), jnp.int32),
            pltpu.VMEM((SB,), jnp.float32),
            pltpu.VMEM((SB,), jnp.int32),
            pltpu.VMEM((SB,), jnp.float32),
            pltpu.VMEM((2, CH), jnp.int32),
            pltpu.VMEM((2, CH, D), jnp.float32),
            pltpu.VMEM((2, CH, D), jnp.float32),
            pltpu.VMEM_SHARED((ROWS_A, D), jnp.float32),
            pltpu.SemaphoreType.DMA,
            pltpu.SemaphoreType.DMA,
            pltpu.SemaphoreType.DMA,
            pltpu.SemaphoreType.DMA,
            pltpu.SemaphoreType.DMA,
            pltpu.SemaphoreType.DMA,
            pltpu.SemaphoreType.DMA,
        ],
    )
    return f(ent_emb, rel_emb, src, dst, etype, edir, enorm)


def _tc_body(acca_ref, accb_ref, w_ref, rel_ref, wrel_ref, gam_ref, bet_ref,
             out_ref, relout_ref):
    acc0 = acca_ref[0, :N_NODES, :]
    acc1 = acca_ref[1, :N_NODES, :]
    acc2 = jnp.concatenate([accb_ref[0, :HALF, :], accb_ref[1, :HALF, :]],
                           axis=0)
    y = jnp.dot(acc0, w_ref[0], preferred_element_type=jnp.float32)
    y = y + jnp.dot(acc1, w_ref[1], preferred_element_type=jnp.float32)
    y = y + jnp.dot(acc2, w_ref[2], preferred_element_type=jnp.float32)
    mean = jnp.mean(y, axis=0, keepdims=True)
    var = jnp.mean((y - mean) * (y - mean), axis=0, keepdims=True)
    yn = (y - mean) * lax.rsqrt(var + 1e-5) * gam_ref[...] + bet_ref[...]
    out_ref[...] = jnp.maximum(yn, 0.0)
    relout_ref[...] = jnp.maximum(
        jnp.dot(rel_ref[...], wrel_ref[...],
                preferred_element_type=jnp.float32), 0.0)


def _tc_finish(acca, accb, w_stack, rel_emb, w_rel_t, gamma, beta):
    return pl.pallas_call(
        _tc_body,
        out_shape=(jax.ShapeDtypeStruct((N_NODES, D), jnp.float32),
                   jax.ShapeDtypeStruct((N_RELS, D), jnp.float32)),
    )(acca, accb, w_stack, rel_emb, w_rel_t, gamma, beta)


def kernel(ent_emb, rel_emb, edge_norm, W_in, W_out, W_loop, W_rel, gamma,
           beta, edge_index, edge_type, edge_dir):
    src = edge_index[0].astype(jnp.int32)
    dst = edge_index[1].astype(jnp.int32)
    etype = edge_type.astype(jnp.int32)
    edir = edge_dir.astype(jnp.int32)

    acca, accb = _sc_aggregate(ent_emb, rel_emb, src, dst, etype, edir,
                               edge_norm)

    w_stack = jnp.stack([W_out.T, W_in.T, W_loop.T])
    out, rel_out = _tc_finish(acca, accb, w_stack, rel_emb, W_rel.T,
                              gamma.reshape(1, D), beta.reshape(1, D))
    return (out, rel_out)
